# baseline (speedup 1.0000x reference)
import functools

import jax
import jax.numpy as jnp
from jax.experimental import pallas as pl
from jax.experimental.pallas import tpu as pltpu
from jax.experimental.pallas import tpu_sc as plsc

N = 50000
E = 800000
H = 64
TW = 80
BN = 2000
BE = 4000

_NC, _NS, _NW = 2, 16, 32
_CHUNK = 128
_NCHUNK = E // _CHUNK
_GITER = -(-_NCHUNK // _NW)
_SITER = -(-_NCHUNK // _NS)
_RPT = N // _NS

_HIGH = jax.lax.Precision.HIGHEST


def _silu(x):
    return x * jax.nn.sigmoid(x)


def _dot(a, b):
    return jnp.dot(a, b, preferred_element_type=jnp.float32, precision=_HIGH)


def _dot1(a, b):
    return jnp.dot(a.astype(jnp.bfloat16), b.astype(jnp.bfloat16),
                   preferred_element_type=jnp.float32)


def _emb_body(x_ref, coord_ref, tbl_ref, bin_ref, wa_ref, wb_ref,
              h_ref, c4_ref, ts_ref, td_ref):
    x = x_ref[...]
    mx = jnp.max(x, axis=1, keepdims=True)
    it = jax.lax.broadcasted_iota(jnp.int32, x.shape, 1)
    idx = jnp.min(jnp.where(x == mx, it, x.shape[1]), axis=1, keepdims=True)
    onehot = (it == idx).astype(jnp.float32)
    h = _dot(onehot, tbl_ref[...]) + bin_ref[...]
    h_ref[...] = h
    c16 = jnp.pad(coord_ref[...], ((0, 0), (0, 13)))
    c4_ref[...] = c16[:, :4]
    ts_ref[...] = jnp.concatenate([_dot(h, wa_ref[...]), c16], axis=1)
    td_ref[...] = jnp.concatenate([_dot(h, wb_ref[...]), c16], axis=1)


def _emb_call(x, coord, eff_tbl, b_in, wa, wb):
    grid = (N // BN,)
    return pl.pallas_call(
        _emb_body,
        grid=grid,
        in_specs=[
            pl.BlockSpec((BN, 24), lambda i: (i, 0)),
            pl.BlockSpec((BN, 3), lambda i: (i, 0)),
            pl.BlockSpec((24, H), lambda i: (0, 0)),
            pl.BlockSpec((1, H), lambda i: (0, 0)),
            pl.BlockSpec((H, H), lambda i: (0, 0)),
            pl.BlockSpec((H, H), lambda i: (0, 0)),
        ],
        out_specs=[
            pl.BlockSpec((BN, H), lambda i: (i, 0)),
            pl.BlockSpec((BN, 4), lambda i: (i, 0)),
            pl.BlockSpec((BN, TW), lambda i: (i, 0)),
            pl.BlockSpec((BN, TW), lambda i: (i, 0)),
        ],
        out_shape=[
            jax.ShapeDtypeStruct((N, H), jnp.float32),
            jax.ShapeDtypeStruct((N, 4), jnp.float32),
            jax.ShapeDtypeStruct((N, TW), jnp.float32),
            jax.ShapeDtypeStruct((N, TW), jnp.float32),
        ],
    )(x, coord, eff_tbl, b_in, wa, wb)


def _sc_gather_body(ts, td, rowr, colr, o_out,
                    idx_r, idx_c, a_v, b_v, o_v, sem_a, sem_b):
    c = jax.lax.axis_index("c")
    s = jax.lax.axis_index("s")
    wid = s * _NC + c

    def chunk(j, carry):
        cw = wid + j * _NW

        @pl.when(cw < _NCHUNK)
        def _():
            e0 = cw * _CHUNK
            pltpu.sync_copy(rowr.at[pl.ds(e0, _CHUNK)], idx_r)
            pltpu.sync_copy(colr.at[pl.ds(e0, _CHUNK)], idx_c)
            da = pltpu.async_copy(ts.at[idx_r], a_v, sem_a)
            db = pltpu.async_copy(td.at[idx_c], b_v, sem_b)
            da.wait()
            db.wait()

            def rowbody(i, cy):
                for jj in range(4):
                    sl = pl.ds(jj * 16, 16)
                    o_v[i, sl] = a_v[i, sl] + b_v[i, sl]
                sl = pl.ds(H, 16)
                o_v[i, sl] = a_v[i, sl] - b_v[i, sl]
                return cy

            jax.lax.fori_loop(0, _CHUNK, rowbody, 0)
            pltpu.sync_copy(o_v, o_out.at[pl.ds(e0, _CHUNK)])

        return carry

    jax.lax.fori_loop(0, _GITER, chunk, 0)


def _sc_gather(ts, td, row, col):
    f32 = jnp.float32
    mesh = plsc.VectorSubcoreMesh(core_axis_name="c", subcore_axis_name="s")
    return pl.kernel(
        _sc_gather_body,
        out_type=jax.ShapeDtypeStruct((E, TW), f32),
        mesh=mesh,
        compiler_params=pltpu.CompilerParams(use_tc_tiling_on_sc=False),
        scratch_types=[
            pltpu.VMEM((_CHUNK,), jnp.int32),
            pltpu.VMEM((_CHUNK,), jnp.int32),
            pltpu.VMEM((_CHUNK, TW), f32),
            pltpu.VMEM((_CHUNK, TW), f32),
            pltpu.VMEM((_CHUNK, TW), f32),
            pltpu.SemaphoreType.DMA,
            pltpu.SemaphoreType.DMA,
        ],
    )(ts, td, row, col)


def _edge_body(gd_ref, ea_ref, wd_ref, wc_ref, be1_ref, we2_ref,
               be2_ref, wc1_ref, bc1_ref, wc2_ref, mlo_ref, mhi_ref, t4_ref):
    gd = gd_ref[...]
    g = gd[:, :H]
    dr = gd[:, H:H + 4]
    radial = jnp.sum(dr * dr, axis=1, keepdims=True)
    pre = g + radial * wc_ref[...] + be1_ref[...]
    pre = pre + _dot(ea_ref[...], wd_ref[...])
    m1 = _silu(pre)
    m2 = _silu(_dot(m1, we2_ref[...]) + be2_ref[...])
    mlo_ref[...] = m2[:, :H // 2]
    mhi_ref[...] = m2[:, H // 2:]
    t = _dot1(_silu(_dot1(m2, wc1_ref[...]) + bc1_ref[...]), wc2_ref[...])
    t4 = dr * t
    it = jax.lax.broadcasted_iota(jnp.int32, t4.shape, 1)
    t4_ref[...] = jnp.where(it == 3, 1.0, t4)


def _edge_call(gd, edge_attr, wd, wc, be1, we2, be2, wc1, bc1, wc2):
    grid = (E // BE,)
    return pl.pallas_call(
        _edge_body,
        grid=grid,
        in_specs=[
            pl.BlockSpec((BE, TW), lambda i: (i, 0)),
            pl.BlockSpec((BE, 16), lambda i: (i, 0)),
            pl.BlockSpec((16, H), lambda i: (0, 0)),
            pl.BlockSpec((1, H), lambda i: (0, 0)),
            pl.BlockSpec((1, H), lambda i: (0, 0)),
            pl.BlockSpec((H, H), lambda i: (0, 0)),
            pl.BlockSpec((1, H), lambda i: (0, 0)),
            pl.BlockSpec((H, H), lambda i: (0, 0)),
            pl.BlockSpec((1, H), lambda i: (0, 0)),
            pl.BlockSpec((H, 1), lambda i: (0, 0)),
        ],
        out_specs=[
            pl.BlockSpec((BE, H // 2), lambda i: (i, 0)),
            pl.BlockSpec((BE, H // 2), lambda i: (i, 0)),
            pl.BlockSpec((BE, 4), lambda i: (i, 0)),
        ],
        out_shape=[
            jax.ShapeDtypeStruct((E, H // 2), jnp.float32),
            jax.ShapeDtypeStruct((E, H // 2), jnp.float32),
            jax.ShapeDtypeStruct((E, 4), jnp.float32),
        ],
    )(gd, edge_attr, wd, wc, be1, we2, be2, wc1, bc1, wc2)


def _sc_scatter_body(mlo, mhi, t4, rowr, zeros32, zeros4,
                     agg_lo, agg_hi, s4_out,
                     idx_v, m_v, t4_v, acc, s4acc):
    c = jax.lax.axis_index("c")
    s = jax.lax.axis_index("s")
    r0 = s * _RPT

    pltpu.sync_copy(zeros32, acc.at[pl.ds(r0, _RPT)])

    @pl.when(c == 1)
    def _():
        pltpu.sync_copy(zeros4, s4acc.at[pl.ds(r0, _RPT)])

    plsc.subcore_barrier()

    def chunk(j, carry):
        cw = s + j * _NS

        @pl.when(cw < _NCHUNK)
        def _():
            e0 = cw * _CHUNK
            pltpu.sync_copy(rowr.at[pl.ds(e0, _CHUNK)], idx_v)

            @pl.when(c == 0)
            def _():
                pltpu.sync_copy(mlo.at[pl.ds(e0, _CHUNK)], m_v)
                pltpu.sync_copy(m_v, acc.at[idx_v], add=True)

            @pl.when(c == 1)
            def _():
                pltpu.sync_copy(mhi.at[pl.ds(e0, _CHUNK)], m_v)
                pltpu.sync_copy(m_v, acc.at[idx_v], add=True)
                pltpu.sync_copy(t4.at[pl.ds(e0, _CHUNK)], t4_v)
                pltpu.sync_copy(t4_v, s4acc.at[idx_v], add=True)

        return carry

    jax.lax.fori_loop(0, _SITER, chunk, 0)
    plsc.subcore_barrier()

    @pl.when(c == 0)
    def _():
        pltpu.sync_copy(acc.at[pl.ds(r0, _RPT)], agg_lo.at[pl.ds(r0, _RPT)])

    @pl.when(c == 1)
    def _():
        pltpu.sync_copy(acc.at[pl.ds(r0, _RPT)], agg_hi.at[pl.ds(r0, _RPT)])
        pltpu.sync_copy(s4acc.at[pl.ds(r0, _RPT)],
                        s4_out.at[pl.ds(r0, _RPT)])


def _sc_scatter(mlo, mhi, t4, row, zeros32, zeros4):
    f32 = jnp.float32
    mesh = plsc.VectorSubcoreMesh(core_axis_name="c", subcore_axis_name="s")
    return pl.kernel(
        _sc_scatter_body,
        out_type=[
            jax.ShapeDtypeStruct((N, H // 2), f32),
            jax.ShapeDtypeStruct((N, H // 2), f32),
            jax.ShapeDtypeStruct((N, 4), f32),
        ],
        mesh=mesh,
        compiler_params=pltpu.CompilerParams(use_tc_tiling_on_sc=False),
        scratch_types=[
            pltpu.VMEM((_CHUNK,), jnp.int32),
            pltpu.VMEM((_CHUNK, H // 2), f32),
            pltpu.VMEM((_CHUNK, 4), f32),
            pltpu.VMEM_SHARED((N, H // 2), f32),
            pltpu.VMEM_SHARED((N, 4), f32),
        ],
    )(mlo, mhi, t4, row, zeros32, zeros4)


def _node_body(last, h_ref, alo_ref, ahi_ref, s4_ref, c4_ref, wnh_ref,
               wnl_ref, wnr_ref, bn1_ref, wn2_ref, bn2_ref, wx_ref, bx_ref,
               *out_refs):
    h = h_ref[...]
    u = _silu(_dot(h, wnh_ref[...]) + _dot(alo_ref[...], wnl_ref[...])
              + _dot(ahi_ref[...], wnr_ref[...]) + bn1_ref[...])
    hn = h + _dot(u, wn2_ref[...]) + bn2_ref[...]
    if last:
        out_ref, = out_refs
        out_ref[...] = jax.nn.sigmoid(_dot(hn, wx_ref[...]) + bx_ref[...])
    else:
        hn_ref, c4n_ref, ts_ref, td_ref = out_refs
        hn_ref[...] = hn
        s4 = s4_ref[...]
        cnt = jnp.maximum(s4[:, 3:4], 1.0)
        c4 = c4_ref[...] + s4 / cnt
        it = jax.lax.broadcasted_iota(jnp.int32, c4.shape, 1)
        c4n = jnp.where(it == 3, 0.0, c4)
        c4n_ref[...] = c4n
        c16 = jnp.pad(c4n, ((0, 0), (0, 12)))
        ts_ref[...] = jnp.concatenate([_dot(hn, wx_ref[..., :H]), c16], axis=1)
        td_ref[...] = jnp.concatenate([_dot(hn, wx_ref[..., H:]), c16], axis=1)


def _node_call(last, h, agg_lo, agg_hi, s4, c4, wnh, wnl, wnr, bn1, wn2, bn2,
               wx, bx):
    grid = (N // BN,)
    nspec = pl.BlockSpec((BN, H), lambda i: (i, 0))
    aspec = pl.BlockSpec((BN, H // 2), lambda i: (i, 0))
    sspec = pl.BlockSpec((BN, 4), lambda i: (i, 0))
    wspec = pl.BlockSpec((H, H), lambda i: (0, 0))
    hwspec = pl.BlockSpec((H // 2, H), lambda i: (0, 0))
    bspec = pl.BlockSpec((1, H), lambda i: (0, 0))
    if last:
        wx_spec = pl.BlockSpec((H, 1), lambda i: (0, 0))
        bx_spec = pl.BlockSpec((1, 1), lambda i: (0, 0))
        out_specs = [pl.BlockSpec((BN, 1), lambda i: (i, 0))]
        out_shape = [jax.ShapeDtypeStruct((N, 1), jnp.float32)]
    else:
        wx_spec = pl.BlockSpec((H, 2 * H), lambda i: (0, 0))
        bx_spec = pl.BlockSpec((1, H), lambda i: (0, 0))
        tspec = pl.BlockSpec((BN, TW), lambda i: (i, 0))
        out_specs = [nspec, sspec, tspec, tspec]
        out_shape = [
            jax.ShapeDtypeStruct((N, H), jnp.float32),
            jax.ShapeDtypeStruct((N, 4), jnp.float32),
            jax.ShapeDtypeStruct((N, TW), jnp.float32),
            jax.ShapeDtypeStruct((N, TW), jnp.float32),
        ]
    return pl.pallas_call(
        functools.partial(_node_body, last),
        grid=grid,
        in_specs=[nspec, aspec, aspec, sspec, sspec, wspec, hwspec, hwspec,
                  bspec, wspec, bspec, wx_spec, bx_spec],
        out_specs=out_specs,
        out_shape=out_shape,
    )(h, agg_lo, agg_hi, s4, c4, wnh, wnl, wnr, bn1, wn2, bn2, wx, bx)


def kernel(x, coord, edge_attr, edge_index, emb_table, W_in, b_in, We1, be1,
           We2, be2, Wn1, bn1, Wn2, bn2, Wc1, bc1, Wc2, W_out, b_out,
           W_pred, b_pred):
    f32 = jnp.float32
    row = edge_index[0]
    col = edge_index[1]
    zeros32 = jnp.zeros((_RPT, H // 2), f32)
    zeros4 = jnp.zeros((_RPT, 4), f32)
    eff_tbl = emb_table @ W_in
    w_eff = W_out @ W_pred
    b_eff = (b_out @ W_pred + b_pred).reshape(1, 1)

    L = We1.shape[0]
    wa = [We1[i, :H] for i in range(L)]
    wb = [We1[i, H:2 * H] for i in range(L)]
    wc = [We1[i, 2 * H:2 * H + 1] for i in range(L)]
    wd = [We1[i, 2 * H + 1:] for i in range(L)]
    wnh = [Wn1[i, :H] for i in range(L)]
    wnl = [Wn1[i, H:H + H // 2] for i in range(L)]
    wnr = [Wn1[i, H + H // 2:] for i in range(L)]

    h, c4, ts, td = _emb_call(x, coord, eff_tbl.astype(f32),
                              b_in.reshape(1, H), wa[0], wb[0])
    for i in range(L):
        gd = _sc_gather(ts, td, row, col)
        mlo, mhi, t4 = _edge_call(gd, edge_attr, wd[i], wc[i],
                                  be1[i].reshape(1, H), We2[i],
                                  be2[i].reshape(1, H), Wc1[i],
                                  bc1[i].reshape(1, H), Wc2[i])
        agg_lo, agg_hi, s4 = _sc_scatter(mlo, mhi, t4, row, zeros32, zeros4)
        last = i == L - 1
        if last:
            out, = _node_call(True, h, agg_lo, agg_hi, s4, c4, wnh[i],
                              wnl[i], wnr[i], bn1[i].reshape(1, H), Wn2[i],
                              bn2[i].reshape(1, H), w_eff, b_eff)
        else:
            wx = jnp.concatenate([We1[i + 1, :H], We1[i + 1, H:2 * H]],
                                 axis=1)
            h, c4, ts, td = _node_call(False, h, agg_lo, agg_hi, s4, c4,
                                       wnh[i], wnl[i], wnr[i],
                                       bn1[i].reshape(1, H), Wn2[i],
                                       bn2[i].reshape(1, H), wx,
                                       bn2[i].reshape(1, H))
    return out


# speedup vs baseline: 1.1622x; 1.1622x over previous
import functools

import jax
import jax.numpy as jnp
from jax.experimental import pallas as pl
from jax.experimental.pallas import tpu as pltpu
from jax.experimental.pallas import tpu_sc as plsc

N = 50000
E = 800000
H = 64
TW = 80
BN = 2000
BE = 4000

_NC, _NS, _NW = 2, 16, 32
_CHUNK = 128
_NCHUNK = E // _CHUNK
_GITER = -(-_NCHUNK // _NW)
_SITER = -(-_NCHUNK // _NS)
_RPT = N // _NS

_HIGH = jax.lax.Precision.HIGHEST


def _silu(x):
    return x * jax.nn.sigmoid(x)


def _dot(a, b):
    return jnp.dot(a, b, preferred_element_type=jnp.float32, precision=_HIGH)


def _split_bf16(a):
    ai = jax.lax.bitcast_convert_type(a, jnp.int32)
    ahi = jax.lax.bitcast_convert_type(
        jax.lax.bitwise_and(ai, jnp.int32(-65536)), jnp.float32)
    return ahi.astype(jnp.bfloat16), (a - ahi).astype(jnp.bfloat16)


def _dot3(a, b):
    ahi, alo = _split_bf16(a)
    bhi, blo = _split_bf16(b)

    def d(u, v):
        return jnp.dot(u, v, preferred_element_type=jnp.float32)

    return d(ahi, bhi) + (d(ahi, blo) + d(alo, bhi))


def _emb_body(x_ref, coord_ref, tbl_ref, bin_ref, wa_ref, wb_ref,
              h_ref, c4_ref, ts_ref, td_ref):
    x = x_ref[...]
    mx = jnp.max(x, axis=1, keepdims=True)
    it = jax.lax.broadcasted_iota(jnp.int32, x.shape, 1)
    idx = jnp.min(jnp.where(x == mx, it, x.shape[1]), axis=1, keepdims=True)
    onehot = (it == idx).astype(jnp.float32)
    h = _dot(onehot, tbl_ref[...]) + bin_ref[...]
    h_ref[...] = h
    c16 = jnp.pad(coord_ref[...], ((0, 0), (0, 13)))
    c4_ref[...] = c16[:, :4]
    ts_ref[...] = jnp.concatenate([_dot(h, wa_ref[...]), c16], axis=1)
    td_ref[...] = jnp.concatenate([_dot(h, wb_ref[...]), c16], axis=1)


def _emb_call(x, coord, eff_tbl, b_in, wa, wb):
    grid = (N // BN,)
    return pl.pallas_call(
        _emb_body,
        grid=grid,
        in_specs=[
            pl.BlockSpec((BN, 24), lambda i: (i, 0)),
            pl.BlockSpec((BN, 3), lambda i: (i, 0)),
            pl.BlockSpec((24, H), lambda i: (0, 0)),
            pl.BlockSpec((1, H), lambda i: (0, 0)),
            pl.BlockSpec((H, H), lambda i: (0, 0)),
            pl.BlockSpec((H, H), lambda i: (0, 0)),
        ],
        out_specs=[
            pl.BlockSpec((BN, H), lambda i: (i, 0)),
            pl.BlockSpec((BN, 4), lambda i: (i, 0)),
            pl.BlockSpec((BN, TW), lambda i: (i, 0)),
            pl.BlockSpec((BN, TW), lambda i: (i, 0)),
        ],
        out_shape=[
            jax.ShapeDtypeStruct((N, H), jnp.float32),
            jax.ShapeDtypeStruct((N, 4), jnp.float32),
            jax.ShapeDtypeStruct((N, TW), jnp.float32),
            jax.ShapeDtypeStruct((N, TW), jnp.float32),
        ],
    )(x, coord, eff_tbl, b_in, wa, wb)


def _sc_gather_body(ts, td, rowr, colr, o_out,
                    idx_r, idx_c, a_v, b_v, o_v, sem_a, sem_b):
    c = jax.lax.axis_index("c")
    s = jax.lax.axis_index("s")
    wid = s * _NC + c

    def chunk(j, carry):
        cw = wid + j * _NW

        @pl.when(cw < _NCHUNK)
        def _():
            e0 = cw * _CHUNK
            pltpu.sync_copy(rowr.at[pl.ds(e0, _CHUNK)], idx_r)
            pltpu.sync_copy(colr.at[pl.ds(e0, _CHUNK)], idx_c)
            da = pltpu.async_copy(ts.at[idx_r], a_v, sem_a)
            db = pltpu.async_copy(td.at[idx_c], b_v, sem_b)
            da.wait()
            db.wait()

            def rowbody(i, cy):
                for jj in range(4):
                    sl = pl.ds(jj * 16, 16)
                    o_v[i, sl] = a_v[i, sl] + b_v[i, sl]
                sl = pl.ds(H, 16)
                o_v[i, sl] = a_v[i, sl] - b_v[i, sl]
                return cy

            jax.lax.fori_loop(0, _CHUNK, rowbody, 0)
            pltpu.sync_copy(o_v, o_out.at[pl.ds(e0, _CHUNK)])

        return carry

    jax.lax.fori_loop(0, _GITER, chunk, 0)


def _sc_gather(ts, td, row, col):
    f32 = jnp.float32
    mesh = plsc.VectorSubcoreMesh(core_axis_name="c", subcore_axis_name="s")
    return pl.kernel(
        _sc_gather_body,
        out_type=jax.ShapeDtypeStruct((E, TW), f32),
        mesh=mesh,
        compiler_params=pltpu.CompilerParams(use_tc_tiling_on_sc=False),
        scratch_types=[
            pltpu.VMEM((_CHUNK,), jnp.int32),
            pltpu.VMEM((_CHUNK,), jnp.int32),
            pltpu.VMEM((_CHUNK, TW), f32),
            pltpu.VMEM((_CHUNK, TW), f32),
            pltpu.VMEM((_CHUNK, TW), f32),
            pltpu.SemaphoreType.DMA,
            pltpu.SemaphoreType.DMA,
        ],
    )(ts, td, row, col)


def _edge_body(gd_ref, ea_ref, wd_ref, wc_ref, be1_ref, we2_ref,
               be2_ref, wc1_ref, bc1_ref, wc2_ref, mlo_ref, mhi_ref, t4_ref):
    gd = gd_ref[...]
    g = gd[:, :H]
    dr = gd[:, H:H + 4]
    radial = jnp.sum(dr * dr, axis=1, keepdims=True)
    pre = g + radial * wc_ref[...] + be1_ref[...]
    pre = pre + _dot3(ea_ref[...], wd_ref[...])
    m1 = _silu(pre)
    m2 = _silu(_dot3(m1, we2_ref[...]) + be2_ref[...])
    mlo_ref[...] = m2[:, :H // 2]
    mhi_ref[...] = m2[:, H // 2:]
    u = _silu(_dot3(m2, wc1_ref[...]) + bc1_ref[...])
    t = jnp.sum(u * wc2_ref[...], axis=1, keepdims=True)
    t4 = dr * t
    it = jax.lax.broadcasted_iota(jnp.int32, t4.shape, 1)
    t4_ref[...] = jnp.where(it == 3, 1.0, t4)


def _edge_call(gd, edge_attr, wd, wc, be1, we2, be2, wc1, bc1, wc2):
    grid = (E // BE,)
    return pl.pallas_call(
        _edge_body,
        grid=grid,
        in_specs=[
            pl.BlockSpec((BE, TW), lambda i: (i, 0)),
            pl.BlockSpec((BE, 16), lambda i: (i, 0)),
            pl.BlockSpec((16, H), lambda i: (0, 0)),
            pl.BlockSpec((1, H), lambda i: (0, 0)),
            pl.BlockSpec((1, H), lambda i: (0, 0)),
            pl.BlockSpec((H, H), lambda i: (0, 0)),
            pl.BlockSpec((1, H), lambda i: (0, 0)),
            pl.BlockSpec((H, H), lambda i: (0, 0)),
            pl.BlockSpec((1, H), lambda i: (0, 0)),
            pl.BlockSpec((1, H), lambda i: (0, 0)),
        ],
        out_specs=[
            pl.BlockSpec((BE, H // 2), lambda i: (i, 0)),
            pl.BlockSpec((BE, H // 2), lambda i: (i, 0)),
            pl.BlockSpec((BE, 4), lambda i: (i, 0)),
        ],
        out_shape=[
            jax.ShapeDtypeStruct((E, H // 2), jnp.float32),
            jax.ShapeDtypeStruct((E, H // 2), jnp.float32),
            jax.ShapeDtypeStruct((E, 4), jnp.float32),
        ],
    )(gd, edge_attr, wd, wc, be1, we2, be2, wc1, bc1, wc2)


def _sc_scatter_body(mlo, mhi, t4, rowr, zeros32, zeros4,
                     agg_lo, agg_hi, s4_out,
                     idx_v, m_v, t4_v, acc, s4acc):
    c = jax.lax.axis_index("c")
    s = jax.lax.axis_index("s")
    r0 = s * _RPT

    pltpu.sync_copy(zeros32, acc.at[pl.ds(r0, _RPT)])

    @pl.when(c == 1)
    def _():
        pltpu.sync_copy(zeros4, s4acc.at[pl.ds(r0, _RPT)])

    plsc.subcore_barrier()

    def chunk(j, carry):
        cw = s + j * _NS

        @pl.when(cw < _NCHUNK)
        def _():
            e0 = cw * _CHUNK
            pltpu.sync_copy(rowr.at[pl.ds(e0, _CHUNK)], idx_v)

            @pl.when(c == 0)
            def _():
                pltpu.sync_copy(mlo.at[pl.ds(e0, _CHUNK)], m_v)
                pltpu.sync_copy(m_v, acc.at[idx_v], add=True)

            @pl.when(c == 1)
            def _():
                pltpu.sync_copy(mhi.at[pl.ds(e0, _CHUNK)], m_v)
                pltpu.sync_copy(m_v, acc.at[idx_v], add=True)
                pltpu.sync_copy(t4.at[pl.ds(e0, _CHUNK)], t4_v)
                pltpu.sync_copy(t4_v, s4acc.at[idx_v], add=True)

        return carry

    jax.lax.fori_loop(0, _SITER, chunk, 0)
    plsc.subcore_barrier()

    @pl.when(c == 0)
    def _():
        pltpu.sync_copy(acc.at[pl.ds(r0, _RPT)], agg_lo.at[pl.ds(r0, _RPT)])

    @pl.when(c == 1)
    def _():
        pltpu.sync_copy(acc.at[pl.ds(r0, _RPT)], agg_hi.at[pl.ds(r0, _RPT)])
        pltpu.sync_copy(s4acc.at[pl.ds(r0, _RPT)],
                        s4_out.at[pl.ds(r0, _RPT)])


def _sc_scatter(mlo, mhi, t4, row, zeros32, zeros4):
    f32 = jnp.float32
    mesh = plsc.VectorSubcoreMesh(core_axis_name="c", subcore_axis_name="s")
    return pl.kernel(
        _sc_scatter_body,
        out_type=[
            jax.ShapeDtypeStruct((N, H // 2), f32),
            jax.ShapeDtypeStruct((N, H // 2), f32),
            jax.ShapeDtypeStruct((N, 4), f32),
        ],
        mesh=mesh,
        compiler_params=pltpu.CompilerParams(use_tc_tiling_on_sc=False),
        scratch_types=[
            pltpu.VMEM((_CHUNK,), jnp.int32),
            pltpu.VMEM((_CHUNK, H // 2), f32),
            pltpu.VMEM((_CHUNK, 4), f32),
            pltpu.VMEM_SHARED((N, H // 2), f32),
            pltpu.VMEM_SHARED((N, 4), f32),
        ],
    )(mlo, mhi, t4, row, zeros32, zeros4)


def _node_body(last, h_ref, alo_ref, ahi_ref, s4_ref, c4_ref, wnh_ref,
               wnl_ref, wnr_ref, bn1_ref, wn2_ref, bn2_ref, wx_ref, bx_ref,
               *out_refs):
    h = h_ref[...]
    u = _silu(_dot(h, wnh_ref[...]) + _dot(alo_ref[...], wnl_ref[...])
              + _dot(ahi_ref[...], wnr_ref[...]) + bn1_ref[...])
    hn = h + _dot(u, wn2_ref[...]) + bn2_ref[...]
    if last:
        out_ref, = out_refs
        out_ref[...] = jax.nn.sigmoid(_dot(hn, wx_ref[...]) + bx_ref[...])
    else:
        hn_ref, c4n_ref, ts_ref, td_ref = out_refs
        hn_ref[...] = hn
        s4 = s4_ref[...]
        cnt = jnp.maximum(s4[:, 3:4], 1.0)
        c4 = c4_ref[...] + s4 / cnt
        it = jax.lax.broadcasted_iota(jnp.int32, c4.shape, 1)
        c4n = jnp.where(it == 3, 0.0, c4)
        c4n_ref[...] = c4n
        c16 = jnp.pad(c4n, ((0, 0), (0, 12)))
        ts_ref[...] = jnp.concatenate([_dot(hn, wx_ref[..., :H]), c16], axis=1)
        td_ref[...] = jnp.concatenate([_dot(hn, wx_ref[..., H:]), c16], axis=1)


def _node_call(last, h, agg_lo, agg_hi, s4, c4, wnh, wnl, wnr, bn1, wn2, bn2,
               wx, bx):
    grid = (N // BN,)
    nspec = pl.BlockSpec((BN, H), lambda i: (i, 0))
    aspec = pl.BlockSpec((BN, H // 2), lambda i: (i, 0))
    sspec = pl.BlockSpec((BN, 4), lambda i: (i, 0))
    wspec = pl.BlockSpec((H, H), lambda i: (0, 0))
    hwspec = pl.BlockSpec((H // 2, H), lambda i: (0, 0))
    bspec = pl.BlockSpec((1, H), lambda i: (0, 0))
    if last:
        wx_spec = pl.BlockSpec((H, 1), lambda i: (0, 0))
        bx_spec = pl.BlockSpec((1, 1), lambda i: (0, 0))
        out_specs = [pl.BlockSpec((BN, 1), lambda i: (i, 0))]
        out_shape = [jax.ShapeDtypeStruct((N, 1), jnp.float32)]
    else:
        wx_spec = pl.BlockSpec((H, 2 * H), lambda i: (0, 0))
        bx_spec = pl.BlockSpec((1, H), lambda i: (0, 0))
        tspec = pl.BlockSpec((BN, TW), lambda i: (i, 0))
        out_specs = [nspec, sspec, tspec, tspec]
        out_shape = [
            jax.ShapeDtypeStruct((N, H), jnp.float32),
            jax.ShapeDtypeStruct((N, 4), jnp.float32),
            jax.ShapeDtypeStruct((N, TW), jnp.float32),
            jax.ShapeDtypeStruct((N, TW), jnp.float32),
        ]
    return pl.pallas_call(
        functools.partial(_node_body, last),
        grid=grid,
        in_specs=[nspec, aspec, aspec, sspec, sspec, wspec, hwspec, hwspec,
                  bspec, wspec, bspec, wx_spec, bx_spec],
        out_specs=out_specs,
        out_shape=out_shape,
    )(h, agg_lo, agg_hi, s4, c4, wnh, wnl, wnr, bn1, wn2, bn2, wx, bx)


def kernel(x, coord, edge_attr, edge_index, emb_table, W_in, b_in, We1, be1,
           We2, be2, Wn1, bn1, Wn2, bn2, Wc1, bc1, Wc2, W_out, b_out,
           W_pred, b_pred):
    f32 = jnp.float32
    row = edge_index[0]
    col = edge_index[1]
    zeros32 = jnp.zeros((_RPT, H // 2), f32)
    zeros4 = jnp.zeros((_RPT, 4), f32)
    eff_tbl = emb_table @ W_in
    w_eff = W_out @ W_pred
    b_eff = (b_out @ W_pred + b_pred).reshape(1, 1)

    L = We1.shape[0]
    wa = [We1[i, :H] for i in range(L)]
    wb = [We1[i, H:2 * H] for i in range(L)]
    wc = [We1[i, 2 * H:2 * H + 1] for i in range(L)]
    wd = [We1[i, 2 * H + 1:] for i in range(L)]
    wnh = [Wn1[i, :H] for i in range(L)]
    wnl = [Wn1[i, H:H + H // 2] for i in range(L)]
    wnr = [Wn1[i, H + H // 2:] for i in range(L)]

    h, c4, ts, td = _emb_call(x, coord, eff_tbl.astype(f32),
                              b_in.reshape(1, H), wa[0], wb[0])
    for i in range(L):
        gd = _sc_gather(ts, td, row, col)
        mlo, mhi, t4 = _edge_call(gd, edge_attr, wd[i], wc[i],
                                  be1[i].reshape(1, H), We2[i],
                                  be2[i].reshape(1, H), Wc1[i],
                                  bc1[i].reshape(1, H), Wc2[i].reshape(1, H))
        agg_lo, agg_hi, s4 = _sc_scatter(mlo, mhi, t4, row, zeros32, zeros4)
        last = i == L - 1
        if last:
            out, = _node_call(True, h, agg_lo, agg_hi, s4, c4, wnh[i],
                              wnl[i], wnr[i], bn1[i].reshape(1, H), Wn2[i],
                              bn2[i].reshape(1, H), w_eff, b_eff)
        else:
            wx = jnp.concatenate([We1[i + 1, :H], We1[i + 1, H:2 * H]],
                                 axis=1)
            h, c4, ts, td = _node_call(False, h, agg_lo, agg_hi, s4, c4,
                                       wnh[i], wnl[i], wnr[i],
                                       bn1[i].reshape(1, H), Wn2[i],
                                       bn2[i].reshape(1, H), wx,
                                       bn2[i].reshape(1, H))
    return out
